# baseline (device time: 131495 ns/iter reference)
import numpy as np

import jax
import jax.numpy as jnp
from jax import lax
from jax.experimental import pallas as pl
from jax.experimental.pallas import tpu as pltpu

N_DEV = 16
M_BLK = 512
K_BLK = 512
N_OUT = 4096
W_SUB = 512
N_SUB = K_BLK // W_SUB
N_STEPS = N_DEV * N_SUB
W_BUFS = 2

_QXY = {0: (0, 0), 1: (0, 1), 2: (1, 1), 3: (1, 0)}

N_NEAR = 7
N_CROSS = 8
SND_AHEAD = 2


def _build_tables() -> tuple[np.ndarray, np.ndarray]:
    order = np.zeros((N_DEV, N_DEV - 1), np.int32)
    send = np.zeros((N_DEV, N_DEV - 1), np.int32)
    for j in range(N_DEV):
        zj, qj = divmod(j, 4)
        xj, yj = _QXY[qj]
        rj, side = j % 8, j // 8

        def dist_key(s):
            zs, qs = divmod(s, 4)
            xs, ys = _QXY[qs]
            return (abs(zs - zj), abs(xs - xj) + abs(ys - yj), s)

        near = sorted((s for s in range(N_DEV) if s != j and s // 8 == side),
                      key=dist_key)
        cross_src = sorted((s for s in range(N_DEV) if s // 8 != side),
                           key=lambda s: (rj - s % 8) % 8)
        for n, s in enumerate(near + cross_src):
            order[j, n] = (j - s) % N_DEV
        cross_dst = sorted((s for s in range(N_DEV) if s // 8 != side),
                           key=lambda s: (s % 8 - rj) % 8)
        for m, s in enumerate(near + cross_dst):
            send[j, m] = (s - j) % N_DEV
    return order, send


_ORDER, _SEND = _build_tables()


def kernel(x, w_mat):
    def body(x_ref, w_ref, order_ref, send_ref, out_ref, recv_buf, wbuf,
             send_sems, recv_sems, wsems):
        my = lax.axis_index("i")

        barrier = pltpu.get_barrier_semaphore()
        for d in range(1, N_DEV):
            dst = lax.rem(my + d, N_DEV)
            pl.semaphore_signal(
                barrier, inc=1, device_id=(dst,),
                device_id_type=pl.DeviceIdType.MESH,
            )
        pl.semaphore_wait(barrier, N_DEV - 1)

        def send_rdma(d):
            dst = lax.rem(my + d, N_DEV)
            return pltpu.make_async_remote_copy(
                src_ref=x_ref.at[pl.ds(dst * M_BLK, M_BLK), :],
                dst_ref=recv_buf.at[d],
                send_sem=send_sems.at[d],
                recv_sem=recv_sems.at[d],
                device_id=(dst,),
                device_id_type=pl.DeviceIdType.MESH,
            )

        for m in range(N_NEAR + SND_AHEAD):
            send_rdma(send_ref[my, m]).start()

        recv_buf[0, :, :] = x_ref[pl.ds(my * M_BLK, M_BLK), :]

        def block_offset(n):
            return order_ref[my, n - 1]

        def block_source(n):
            if n == 0:
                return my
            return lax.rem(my - block_offset(n) + N_DEV, N_DEV)

        def w_dma(t):
            n, h = divmod(t, N_SUB)
            s = block_source(n)
            return pltpu.make_async_copy(
                w_ref.at[pl.ds(s * K_BLK + h * W_SUB, W_SUB), :],
                wbuf.at[t % W_BUFS],
                wsems.at[t % W_BUFS],
            )

        for t in range(W_BUFS):
            w_dma(t).start()

        for t in range(N_STEPS):
            n, h = divmod(t, N_SUB)
            if n == 0:
                slot = 0
            else:
                slot = block_offset(n)
                if h == 0:
                    m = N_NEAR + SND_AHEAD - 1 + n
                    if m < N_DEV - 1:
                        send_rdma(send_ref[my, m - SND_AHEAD]).wait_send()
                        send_rdma(send_ref[my, m]).start()
                    send_rdma(slot).wait_recv()
            w_dma(t).wait()
            wb = wbuf[t % W_BUFS].astype(jnp.bfloat16)
            a = recv_buf[slot, :, h * W_SUB:(h + 1) * W_SUB]
            contrib = lax.dot_general(
                a, wb, (((1,), (0,)), ((), ())),
                preferred_element_type=jnp.float32,
            )
            if t == 0:
                out_ref[...] = contrib
            elif t == N_STEPS - 1:
                out_ref[...] = jnp.maximum(out_ref[...] + contrib, 0.0)
            else:
                out_ref[...] += contrib
            if t + W_BUFS < N_STEPS:
                w_dma(t + W_BUFS).start()

        for m in list(range(N_NEAR)) + list(range(N_DEV - 1 - SND_AHEAD, N_DEV - 1)):
            send_rdma(send_ref[my, m]).wait_send()

    xb = x.astype(jnp.bfloat16)
    order = jnp.asarray(_ORDER)
    send = jnp.asarray(_SEND)
    return pl.pallas_call(
        body,
        out_shape=jax.ShapeDtypeStruct((M_BLK, N_OUT), jnp.float32),
        in_specs=[
            pl.BlockSpec(memory_space=pltpu.VMEM),
            pl.BlockSpec(memory_space=pl.ANY),
            pl.BlockSpec(memory_space=pltpu.SMEM),
            pl.BlockSpec(memory_space=pltpu.SMEM),
        ],
        out_specs=pl.BlockSpec(memory_space=pltpu.VMEM),
        scratch_shapes=[
            pltpu.VMEM((N_DEV, M_BLK, K_BLK), jnp.bfloat16),
            pltpu.VMEM((W_BUFS, W_SUB, N_OUT), jnp.float32),
            pltpu.SemaphoreType.DMA((N_DEV,)),
            pltpu.SemaphoreType.DMA((N_DEV,)),
            pltpu.SemaphoreType.DMA((W_BUFS,)),
        ],
        compiler_params=pltpu.CompilerParams(collective_id=0),
    )(xb, w_mat, order, send)


# device time: 127078 ns/iter; 1.0348x vs baseline; 1.0348x over previous
import numpy as np

import jax
import jax.numpy as jnp
from jax import lax
from jax.experimental import pallas as pl
from jax.experimental.pallas import tpu as pltpu

N_DEV = 16
M_BLK = 512
K_BLK = 512
N_OUT = 4096
N_HALF = 2
W_N = N_OUT // N_HALF
N_STEPS = N_DEV * N_HALF
W_BUFS = 2

_QXY = {0: (0, 0), 1: (0, 1), 2: (1, 1), 3: (1, 0)}

N_NEAR = 7
N_CROSS = 8
SND_AHEAD = 2


def _build_tables() -> tuple[np.ndarray, np.ndarray]:
    order = np.zeros((N_DEV, N_DEV - 1), np.int32)
    send = np.zeros((N_DEV, N_DEV - 1), np.int32)
    for j in range(N_DEV):
        zj, qj = divmod(j, 4)
        xj, yj = _QXY[qj]
        rj, side = j % 8, j // 8

        def dist_key(s):
            zs, qs = divmod(s, 4)
            xs, ys = _QXY[qs]
            return (abs(zs - zj), abs(xs - xj) + abs(ys - yj), s)

        near = sorted((s for s in range(N_DEV) if s != j and s // 8 == side),
                      key=dist_key)
        cross_src = sorted((s for s in range(N_DEV) if s // 8 != side),
                           key=lambda s: (rj - s % 8) % 8)
        for n, s in enumerate(near + cross_src):
            order[j, n] = (j - s) % N_DEV
        cross_dst = sorted((s for s in range(N_DEV) if s // 8 != side),
                           key=lambda s: (s % 8 - rj) % 8)
        for m, s in enumerate(near + cross_dst):
            send[j, m] = (s - j) % N_DEV
    return order, send


_ORDER, _SEND = _build_tables()


def kernel(x, w_mat):
    def body(x_ref, w_ref, order_ref, out_ref, recv_buf, send_buf, xstage,
             wbuf, send_sems, recv_sems, xsems, wsems):
        my = lax.axis_index("i")

        barrier = pltpu.get_barrier_semaphore()
        for d in range(1, N_DEV):
            dst = lax.rem(my + d, N_DEV)
            pl.semaphore_signal(
                barrier, inc=1, device_id=(dst,),
                device_id_type=pl.DeviceIdType.MESH,
            )
        pl.semaphore_wait(barrier, N_DEV - 1)

        def send_rdma(d):
            dst = lax.rem(my + d, N_DEV)
            return pltpu.make_async_remote_copy(
                src_ref=send_buf.at[d],
                dst_ref=recv_buf.at[d],
                send_sem=send_sems.at[d],
                recv_sem=recv_sems.at[d],
                device_id=(dst,),
                device_id_type=pl.DeviceIdType.MESH,
            )

        def x_dma(d, slot):
            dst = lax.rem(my + d, N_DEV)
            return pltpu.make_async_copy(
                x_ref.at[pl.ds(dst * M_BLK, M_BLK), :],
                xstage.at[slot],
                xsems.at[slot],
            )

        x_dma(0, 0).start()
        x_dma(0, 0).wait()
        recv_buf[0, :, :] = xstage[0].astype(jnp.bfloat16)
        x_dma(1, 1).start()
        for d in range(1, N_DEV):
            x_dma(d, d % 2).wait()
            if d + 1 < N_DEV:
                x_dma(d + 1, (d + 1) % 2).start()
            send_buf[d, :, :] = xstage[d % 2].astype(jnp.bfloat16)
            send_rdma(d).start()

        def block_offset(n):
            return order_ref[my, n - 1]

        def block_source(n):
            if n == 0:
                return my
            return lax.rem(my - block_offset(n) + N_DEV, N_DEV)

        def w_dma(t):
            n, h = divmod(t, N_HALF)
            s = block_source(n)
            return pltpu.make_async_copy(
                w_ref.at[pl.ds(s * K_BLK, K_BLK), pl.ds(h * W_N, W_N)],
                wbuf.at[t % W_BUFS],
                wsems.at[t % W_BUFS],
            )

        for t in range(W_BUFS):
            w_dma(t).start()

        for t in range(N_STEPS):
            n, h = divmod(t, N_HALF)
            if n == 0:
                slot = 0
            else:
                slot = block_offset(n)
                if h == 0:
                    send_rdma(slot).wait_recv()
            w_dma(t).wait()
            wb = wbuf[t % W_BUFS].astype(jnp.bfloat16)
            a = recv_buf[slot, :, :]
            contrib = lax.dot_general(
                a, wb, (((1,), (0,)), ((), ())),
                preferred_element_type=jnp.float32,
            )
            osl = pl.ds(h * W_N, W_N)
            if n == 0:
                out_ref[:, osl] = contrib
            elif n == N_DEV - 1:
                out_ref[:, osl] = jnp.maximum(out_ref[:, osl] + contrib, 0.0)
            else:
                out_ref[:, osl] += contrib
            if t + W_BUFS < N_STEPS:
                w_dma(t + W_BUFS).start()

        for d in range(1, N_DEV):
            send_rdma(d).wait_send()

    order = jnp.asarray(_ORDER)
    return pl.pallas_call(
        body,
        out_shape=jax.ShapeDtypeStruct((M_BLK, N_OUT), jnp.float32),
        in_specs=[
            pl.BlockSpec(memory_space=pl.ANY),
            pl.BlockSpec(memory_space=pl.ANY),
            pl.BlockSpec(memory_space=pltpu.SMEM),
        ],
        out_specs=pl.BlockSpec(memory_space=pltpu.VMEM),
        scratch_shapes=[
            pltpu.VMEM((N_DEV, M_BLK, K_BLK), jnp.bfloat16),
            pltpu.VMEM((N_DEV, M_BLK, K_BLK), jnp.bfloat16),
            pltpu.VMEM((2, M_BLK, K_BLK), jnp.float32),
            pltpu.VMEM((W_BUFS, K_BLK, W_N), jnp.float32),
            pltpu.SemaphoreType.DMA((N_DEV,)),
            pltpu.SemaphoreType.DMA((N_DEV,)),
            pltpu.SemaphoreType.DMA((2,)),
            pltpu.SemaphoreType.DMA((W_BUFS,)),
        ],
        compiler_params=pltpu.CompilerParams(collective_id=0),
    )(x, w_mat, order)


# device time: 119972 ns/iter; 1.0960x vs baseline; 1.0592x over previous
import numpy as np

import jax
import jax.numpy as jnp
from jax import lax
from jax.experimental import pallas as pl
from jax.experimental.pallas import tpu as pltpu

N_DEV = 16
M_BLK = 512
K_BLK = 512
N_OUT = 4096
W_SUB = 512
N_SUB = K_BLK // W_SUB
N_STEPS = N_DEV * N_SUB
W_BUFS = 2

_QXY = {0: (0, 0), 1: (0, 1), 2: (1, 1), 3: (1, 0)}


def _build_order() -> np.ndarray:
    tbl = np.zeros((N_DEV, N_DEV - 1), np.int32)
    for j in range(N_DEV):
        zj, qj = divmod(j, 4)
        xj, yj = _QXY[qj]

        def key(s):
            zs, qs = divmod(s, 4)
            xs, ys = _QXY[qs]
            return (abs(zs - zj), abs(xs - xj) + abs(ys - yj), s)

        srcs = sorted((s for s in range(N_DEV) if s != j), key=key)
        for n, s in enumerate(srcs):
            tbl[j, n] = (j - s) % N_DEV
    return tbl


_ORDER = _build_order()


def kernel(x, w_mat):
    def body(x_ref, w_ref, order_ref, out_ref, recv_buf, wbuf,
             send_sems, recv_sems, wsems):
        my = lax.axis_index("i")

        barrier = pltpu.get_barrier_semaphore()
        for d in range(1, N_DEV):
            dst = lax.rem(my + d, N_DEV)
            pl.semaphore_signal(
                barrier, inc=1, device_id=(dst,),
                device_id_type=pl.DeviceIdType.MESH,
            )
        pl.semaphore_wait(barrier, N_DEV - 1)

        def a2a_rdma(d):
            dst = lax.rem(my + d, N_DEV)
            return pltpu.make_async_remote_copy(
                src_ref=x_ref.at[pl.ds(dst * M_BLK, M_BLK), :],
                dst_ref=recv_buf.at[d],
                send_sem=send_sems.at[d],
                recv_sem=recv_sems.at[d],
                device_id=(dst,),
                device_id_type=pl.DeviceIdType.MESH,
            )

        for d in range(1, N_DEV):
            a2a_rdma(d).start()

        recv_buf[0, :, :] = x_ref[pl.ds(my * M_BLK, M_BLK), :]

        def block_offset(n):
            return order_ref[my, n - 1]

        def block_source(n):
            if n == 0:
                return my
            return lax.rem(my - block_offset(n) + N_DEV, N_DEV)

        def w_dma(t):
            n, h = divmod(t, N_SUB)
            s = block_source(n)
            return pltpu.make_async_copy(
                w_ref.at[pl.ds(s * K_BLK + h * W_SUB, W_SUB), :],
                wbuf.at[t % W_BUFS],
                wsems.at[t % W_BUFS],
            )

        for t in range(W_BUFS):
            w_dma(t).start()

        for t in range(N_STEPS):
            n, h = divmod(t, N_SUB)
            if n == 0:
                slot = 0
            else:
                slot = block_offset(n)
                if h == 0:
                    a2a_rdma(slot).wait_recv()
            w_dma(t).wait()
            wb = wbuf[t % W_BUFS].astype(jnp.bfloat16)
            a = recv_buf[slot, :, h * W_SUB:(h + 1) * W_SUB]
            contrib = lax.dot_general(
                a, wb, (((1,), (0,)), ((), ())),
                preferred_element_type=jnp.float32,
            )
            if t == 0:
                out_ref[...] = contrib
            elif t == N_STEPS - 1:
                out_ref[...] = jnp.maximum(out_ref[...] + contrib, 0.0)
            else:
                out_ref[...] += contrib
            if t + W_BUFS < N_STEPS:
                w_dma(t + W_BUFS).start()

        for d in range(1, N_DEV):
            a2a_rdma(d).wait_send()

    xb = x.astype(jnp.bfloat16)
    order = jnp.asarray(_ORDER)
    return pl.pallas_call(
        body,
        out_shape=jax.ShapeDtypeStruct((M_BLK, N_OUT), jnp.float32),
        in_specs=[
            pl.BlockSpec(memory_space=pltpu.VMEM),
            pl.BlockSpec(memory_space=pl.ANY),
            pl.BlockSpec(memory_space=pltpu.SMEM),
        ],
        out_specs=pl.BlockSpec(memory_space=pltpu.VMEM),
        scratch_shapes=[
            pltpu.VMEM((N_DEV, M_BLK, K_BLK), jnp.bfloat16),
            pltpu.VMEM((W_BUFS, W_SUB, N_OUT), jnp.float32),
            pltpu.SemaphoreType.DMA((N_DEV,)),
            pltpu.SemaphoreType.DMA((N_DEV,)),
            pltpu.SemaphoreType.DMA((W_BUFS,)),
        ],
        compiler_params=pltpu.CompilerParams(collective_id=0),
    )(xb, w_mat, order)
